# Initial kernel scaffold; baseline (speedup 1.0000x reference)
import functools

import jax
import jax.numpy as jnp
from jax import lax
from jax.experimental import pallas as pl
from jax.experimental.pallas import tpu as pltpu
from jax.experimental.pallas import tpu_sc as plsc

B, L, D, V = 4096, 50, 32, 100001
LAM = 0.01

NC, NS, LN = 2, 16, 16
NW = NC * NS
BW = B // NW
RW = BW * L
CB = 2
CR = CB * L
NCHUNK = BW // CB
RPAD = 128

_mesh = plsc.VectorSubcoreMesh(core_axis_name="c", subcore_axis_name="s")


@functools.partial(
    pl.kernel,
    out_type=(
        jax.ShapeDtypeStruct((B * L,), jnp.float32),
        jax.ShapeDtypeStruct((NW, LN), jnp.float32),
        jax.ShapeDtypeStruct((NW, LN), jnp.float32),
    ),
    mesh=_mesh,
    scratch_types=[
        pltpu.VMEM((NCHUNK, CR), jnp.int32),
        pltpu.VMEM((BW, D), jnp.float32),
        pltpu.VMEM((RPAD, D), jnp.float32),
        pltpu.VMEM((RW + 16,), jnp.float32),
        pltpu.VMEM((LN,), jnp.float32),
        pltpu.VMEM((LN,), jnp.float32),
        pltpu.SemaphoreType.DMA,
    ],
)
def _sc_gather_dot(table_hbm, idx_hbm, u_hbm, pred_hbm, gsq_hbm, usq_hbm,
                   idx_v, u_v, rows_v, pred_v, gsq_v, usq_v, sem):
    wid = lax.axis_index("s") * NC + lax.axis_index("c")
    pltpu.sync_copy(idx_hbm.at[wid], idx_v)
    pltpu.sync_copy(u_hbm.at[pl.ds(wid * BW, BW)], u_v)

    lane = lax.iota(jnp.int32, 16)
    zero = jnp.zeros((16,), jnp.float32)

    def chunk_body(c, sq_acc):
        pltpu.async_copy(table_hbm.at[idx_v.at[c]],
                         rows_v.at[pl.ds(0, CR)], sem).wait()
        for bi in range(CB):
            bj = c * CB + bi
            accs = [zero, zero, zero, zero]
            tail_sq = zero
            for d in range(D):
                u_d = u_v[bj, d]
                d_vec = jnp.full((16,), d, jnp.int32)
                for g in range(4):
                    rid = bi * L + g * 16 + lane
                    e = plsc.load_gather(rows_v, [rid, d_vec])
                    accs[g] = accs[g] + u_d * e
                    if g < 3:
                        sq_acc = sq_acc + e * e
                    else:
                        tail_sq = tail_sq + e * e
            sq_acc = sq_acc + jnp.where(lane < L - 48, tail_sq, 0.0)
            for g in range(4):
                off = c * CR + bi * L + g * 16
                pred_v[pl.ds(off, 16)] = accs[g]
        return sq_acc

    sq = lax.fori_loop(0, NCHUNK, chunk_body, zero)

    def u_body(i, us):
        u0 = u_v[i, pl.ds(0, 16)]
        u1 = u_v[i, pl.ds(16, 16)]
        return us + u0 * u0 + u1 * u1

    us = lax.fori_loop(0, BW, u_body, zero)

    gsq_v[...] = sq
    usq_v[...] = us
    pltpu.sync_copy(pred_v.at[pl.ds(0, RW)], pred_hbm.at[pl.ds(wid * RW, RW)])
    pltpu.sync_copy(gsq_v, gsq_hbm.at[wid])
    pltpu.sync_copy(usq_v, usq_hbm.at[wid])


def _tc_body(p_ref, y_ref, m_ref, gsq_ref, usq_ref, out_ref):
    x = p_ref[...]
    y = y_ref[...]
    m = m_ref[...]
    bce = jnp.maximum(x, 0.0) - x * y + jnp.log1p(jnp.exp(-jnp.abs(x)))
    err = jnp.sum(bce * m)
    gs = jnp.sum(gsq_ref[...])
    us = jnp.sum(usq_ref[...])
    out_ref[0, 0] = err + LAM * (jnp.sqrt(us) + jnp.sqrt(gs))


def kernel(user_embedding_update, item, labels, mdsk, item_embeddings):
    u2d = user_embedding_update.reshape(B, D)
    idx = item.astype(jnp.int32).reshape(NW, NCHUNK, CR)
    pred, gsq, usq = _sc_gather_dot(item_embeddings, idx, u2d)

    p = pred.reshape(1600, 128)
    y = labels.reshape(1600, 128)
    m = mdsk.reshape(1600, 128)
    out = pl.pallas_call(
        _tc_body,
        out_shape=jax.ShapeDtypeStruct((1, 1), jnp.float32),
        out_specs=pl.BlockSpec(memory_space=pltpu.SMEM),
    )(p, y, m, gsq, usq)
    return out.reshape(())


# speedup vs baseline: 3.6555x; 3.6555x over previous
import functools

import jax
import jax.numpy as jnp
from jax import lax
from jax.experimental import pallas as pl
from jax.experimental.pallas import tpu as pltpu
from jax.experimental.pallas import tpu_sc as plsc

B, L, D, V = 4096, 50, 32, 100001
LAM = 0.01

NC, NS, LN = 2, 16, 16
NW = NC * NS
BW = B // NW
RW = BW * L
CB = 2
CR = CB * L
NCHUNK = BW // CB
RPAD = 128

def _sc_body(table_hbm, idx_hbm, u_hbm, pred_hbm, gsq_hbm, usq_hbm,
             idx_v, u_v, rows_v, pred_v, gsq_v, usq_v, sem):
    wid = lax.axis_index("s") * NC + lax.axis_index("c")
    pltpu.sync_copy(idx_hbm.at[wid], idx_v)
    pltpu.sync_copy(u_hbm.at[pl.ds(wid * BW, BW)], u_v)

    lane = lax.iota(jnp.int32, 16)
    zero = jnp.zeros((16,), jnp.float32)

    def chunk_body(c, sq_acc):
        pltpu.async_copy(table_hbm.at[idx_v.at[c]],
                         rows_v.at[pl.ds(0, CR)], sem).wait()
        for bi in range(CB):
            bj = c * CB + bi
            accs = [zero, zero, zero, zero]
            tail_sq = zero
            u0 = u_v[bj, pl.ds(0, 16)]
            u1 = u_v[bj, pl.ds(16, 16)]
            for d in range(D):
                u_d = u0[d] if d < 16 else u1[d - 16]
                d_vec = jnp.full((16,), d, jnp.int32)
                for g in range(4):
                    rid = bi * L + g * 16 + lane
                    e = plsc.load_gather(rows_v, [rid, d_vec])
                    accs[g] = accs[g] + u_d * e
                    if g < 3:
                        sq_acc = sq_acc + e * e
                    else:
                        tail_sq = tail_sq + e * e
            sq_acc = sq_acc + jnp.where(lane < L - 48, tail_sq, 0.0)
            for g in range(4):
                off = c * CR + bi * L + g * 16
                pred_v[pl.ds(off, 16)] = accs[g]
        return sq_acc

    sq = lax.fori_loop(0, NCHUNK, chunk_body, zero)

    def u_body(i, us):
        u0 = u_v[i, pl.ds(0, 16)]
        u1 = u_v[i, pl.ds(16, 16)]
        return us + u0 * u0 + u1 * u1

    us = lax.fori_loop(0, BW, u_body, zero)

    gsq_v[...] = sq
    usq_v[...] = us
    pltpu.sync_copy(pred_v.at[pl.ds(0, RW)], pred_hbm.at[pl.ds(wid * RW, RW)])
    pltpu.sync_copy(gsq_v, gsq_hbm.at[wid])
    pltpu.sync_copy(usq_v, usq_hbm.at[wid])


@functools.cache
def _sc_gather_dot():
    mesh = plsc.VectorSubcoreMesh(core_axis_name="c", subcore_axis_name="s",
                                  num_cores=NC, num_subcores=NS)
    return pl.kernel(
        _sc_body,
        out_type=(
            jax.ShapeDtypeStruct((B * L,), jnp.float32),
            jax.ShapeDtypeStruct((NW, LN), jnp.float32),
            jax.ShapeDtypeStruct((NW, LN), jnp.float32),
        ),
        mesh=mesh,
        compiler_params=pltpu.CompilerParams(needs_layout_passes=False,
                                             use_tc_tiling_on_sc=False),
        scratch_types=[
            pltpu.VMEM((NCHUNK, CR), jnp.int32),
            pltpu.VMEM((BW, D), jnp.float32),
            pltpu.VMEM((RPAD, D), jnp.float32),
            pltpu.VMEM((RW + 16,), jnp.float32),
            pltpu.VMEM((LN,), jnp.float32),
            pltpu.VMEM((LN,), jnp.float32),
            pltpu.SemaphoreType.DMA,
        ],
    )


def _tc_body(p_ref, y_ref, m_ref, gsq_ref, usq_ref, out_ref):
    x = p_ref[...]
    y = y_ref[...]
    m = m_ref[...]
    bce = jnp.maximum(x, 0.0) - x * y + jnp.log1p(jnp.exp(-jnp.abs(x)))
    err = jnp.sum(bce * m)
    gs = jnp.sum(gsq_ref[...])
    us = jnp.sum(usq_ref[...])
    out_ref[0, 0] = err + LAM * (jnp.sqrt(us) + jnp.sqrt(gs))


def kernel(user_embedding_update, item, labels, mdsk, item_embeddings):
    u2d = user_embedding_update.reshape(B, D)
    idx = item.astype(jnp.int32).reshape(NW, NCHUNK, CR)
    pred, gsq, usq = _sc_gather_dot()(item_embeddings, idx, u2d)

    p = pred.reshape(1600, 128)
    y = labels.reshape(1600, 128)
    m = mdsk.reshape(1600, 128)
    out = pl.pallas_call(
        _tc_body,
        out_shape=jax.ShapeDtypeStruct((1, 1), jnp.float32),
        out_specs=pl.BlockSpec(memory_space=pltpu.SMEM),
    )(p, y, m, gsq, usq)
    return out.reshape(())


# speedup vs baseline: 4.4999x; 1.2310x over previous
import functools

import jax
import jax.numpy as jnp
from jax import lax
from jax.experimental import pallas as pl
from jax.experimental.pallas import tpu as pltpu
from jax.experimental.pallas import tpu_sc as plsc

B, L, D, V = 4096, 50, 32, 100001
LAM = 0.01

NC, NS, LN = 2, 16, 16
NW = NC * NS
BW = B // NW
RW = BW * L
CB = 2
CR = CB * L
NCHUNK = BW // CB
RPAD = 128

def _sc_body(table_hbm, idx_hbm, u_hbm, pred_hbm, gsq_hbm, usq_hbm,
             idx_v, u_v, rows_v, pred_v, gsq_v, usq_v, sem0, sem1):
    wid = lax.axis_index("s") * NC + lax.axis_index("c")
    pltpu.sync_copy(idx_hbm.at[wid], idx_v)
    pltpu.sync_copy(u_hbm.at[pl.ds(wid * BW, BW)], u_v)

    lane = lax.iota(jnp.int32, 16)
    zero = jnp.zeros((16,), jnp.float32)
    sems = (sem0, sem1)

    def start(c, slot):
        pltpu.async_copy(table_hbm.at[idx_v.at[c]],
                         rows_v.at[slot].at[pl.ds(0, CR)], sems[slot])

    def wait(c, slot):
        pltpu.make_async_copy(table_hbm.at[idx_v.at[c]],
                              rows_v.at[slot].at[pl.ds(0, CR)],
                              sems[slot]).wait()

    def compute(c, slot, sq_acc):
        rows = rows_v.at[slot]
        for bi in range(CB):
            bj = c * CB + bi
            acc_lo = [zero, zero, zero, zero]
            acc_hi = [zero, zero, zero, zero]
            sqg = [zero, zero, zero, zero]
            u0 = u_v[bj, pl.ds(0, 16)]
            u1 = u_v[bj, pl.ds(16, 16)]
            for d in range(D):
                u_d = u0[d] if d < 16 else u1[d - 16]
                d_vec = jnp.full((16,), d, jnp.int32)
                for g in range(4):
                    rid = bi * L + g * 16 + lane
                    e = plsc.load_gather(rows, [rid, d_vec])
                    if d < 16:
                        acc_lo[g] = acc_lo[g] + u_d * e
                    else:
                        acc_hi[g] = acc_hi[g] + u_d * e
                    sqg[g] = sqg[g] + e * e
            sq_acc = (sq_acc + sqg[0] + sqg[1] + sqg[2]
                      + jnp.where(lane < L - 48, sqg[3], 0.0))
            for g in range(4):
                off = c * CR + bi * L + g * 16
                pred_v[pl.ds(off, 16)] = acc_lo[g] + acc_hi[g]
        return sq_acc

    start(0, 0)
    start(1, 1)

    def pair_body(p, sq_acc):
        c0 = 2 * p
        c1 = 2 * p + 1
        wait(c0, 0)
        sq_acc = compute(c0, 0, sq_acc)

        @pl.when(c0 + 2 < NCHUNK)
        def _():
            start(c0 + 2, 0)

        wait(c1, 1)
        sq_acc = compute(c1, 1, sq_acc)

        @pl.when(c1 + 2 < NCHUNK)
        def _():
            start(c1 + 2, 1)

        return sq_acc

    sq = lax.fori_loop(0, NCHUNK // 2, pair_body, zero)

    def u_body(i, us):
        u0 = u_v[i, pl.ds(0, 16)]
        u1 = u_v[i, pl.ds(16, 16)]
        return us + u0 * u0 + u1 * u1

    us = lax.fori_loop(0, BW, u_body, zero)

    gsq_v[...] = sq
    usq_v[...] = us
    pltpu.sync_copy(pred_v.at[pl.ds(0, RW)], pred_hbm.at[pl.ds(wid * RW, RW)])
    pltpu.sync_copy(gsq_v, gsq_hbm.at[wid])
    pltpu.sync_copy(usq_v, usq_hbm.at[wid])


@functools.cache
def _sc_gather_dot():
    mesh = plsc.VectorSubcoreMesh(core_axis_name="c", subcore_axis_name="s",
                                  num_cores=NC, num_subcores=NS)
    return pl.kernel(
        _sc_body,
        out_type=(
            jax.ShapeDtypeStruct((B * L,), jnp.float32),
            jax.ShapeDtypeStruct((NW, LN), jnp.float32),
            jax.ShapeDtypeStruct((NW, LN), jnp.float32),
        ),
        mesh=mesh,
        compiler_params=pltpu.CompilerParams(needs_layout_passes=False,
                                             use_tc_tiling_on_sc=False),
        scratch_types=[
            pltpu.VMEM((NCHUNK, CR), jnp.int32),
            pltpu.VMEM((BW, D), jnp.float32),
            pltpu.VMEM((2, RPAD, D), jnp.float32),
            pltpu.VMEM((RW + 16,), jnp.float32),
            pltpu.VMEM((LN,), jnp.float32),
            pltpu.VMEM((LN,), jnp.float32),
            pltpu.SemaphoreType.DMA,
            pltpu.SemaphoreType.DMA,
        ],
    )


def _tc_body(p_ref, y_ref, m_ref, gsq_ref, usq_ref, out_ref):
    x = p_ref[...]
    y = y_ref[...]
    m = m_ref[...]
    bce = jnp.maximum(x, 0.0) - x * y + jnp.log1p(jnp.exp(-jnp.abs(x)))
    err = jnp.sum(bce * m)
    gs = jnp.sum(gsq_ref[...])
    us = jnp.sum(usq_ref[...])
    out_ref[0, 0] = err + LAM * (jnp.sqrt(us) + jnp.sqrt(gs))


def kernel(user_embedding_update, item, labels, mdsk, item_embeddings):
    u2d = user_embedding_update.reshape(B, D)
    idx = item.astype(jnp.int32).reshape(NW, NCHUNK, CR)
    pred, gsq, usq = _sc_gather_dot()(item_embeddings, idx, u2d)

    p = pred.reshape(1600, 128)
    y = labels.reshape(1600, 128)
    m = mdsk.reshape(1600, 128)
    out = pl.pallas_call(
        _tc_body,
        out_shape=jax.ShapeDtypeStruct((1, 1), jnp.float32),
        out_specs=pl.BlockSpec(memory_space=pltpu.SMEM),
    )(p, y, m, gsq, usq)
    return out.reshape(())
